# baseline (speedup 1.0000x reference)
import functools

import jax
import jax.numpy as jnp
from jax import lax
from jax.experimental import pallas as pl
from jax.experimental.pallas import tpu as pltpu
from jax.experimental.pallas import tpu_sc as plsc

N_NODES = 10000
D = 128
E = 320000
NC = 2
NS = 16
NW = NC * NS
E_PER_TILE = E // NW
CHUNK = 40
FULL = E_PER_TILE // CHUNK
NBUF = 5
TAIL = E_PER_TILE - FULL * CHUNK
ROWS_PER_TILE = 624
ROWS_TAIL = N_NODES - NS * ROWS_PER_TILE
MLP_BLOCK = 2000


def _sc_aggregate(h, ei):
    mesh = plsc.VectorSubcoreMesh(core_axis_name="c", subcore_axis_name="s")

    @functools.partial(
        pl.kernel,
        mesh=mesh,
        out_type=jax.ShapeDtypeStruct((NC, N_NODES, D), jnp.float32),
        scratch_types=[
            pltpu.VMEM((E_PER_TILE,), jnp.int32),
            *[pltpu.VMEM((CHUNK,), jnp.int32) for _ in range(NBUF)],
            *[pltpu.VMEM((CHUNK, D), jnp.float32) for _ in range(NBUF)],
            pltpu.VMEM_SHARED((N_NODES, D), jnp.float32),
            *[pltpu.SemaphoreType.DMA for _ in range(2 * NBUF)],
        ],
    )
    def agg_kernel(h_hbm, ei_hbm, out_hbm, *scr):
        sidx = scr[0]
        ridxs = scr[1:1 + NBUF]
        rowss = scr[1 + NBUF:1 + 2 * NBUF]
        agg = scr[1 + 2 * NBUF]
        gsems = scr[2 + 2 * NBUF:2 + 3 * NBUF]
        rsems = scr[2 + 3 * NBUF:2 + 4 * NBUF]
        rows0 = rowss[0]
        c = lax.axis_index("c")
        s = lax.axis_index("s")
        w = c * NS + s
        ebase = w * E_PER_TILE
        rbase = E + ebase

        pltpu.sync_copy(ei_hbm.at[pl.ds(ebase, E_PER_TILE)], sidx)

        zero = jnp.zeros((16,), jnp.float32)

        def zrow(r, carry):
            for cc in range(D // 16):
                rows0[r, pl.ds(cc * 16, 16)] = zero
            return carry

        lax.fori_loop(0, CHUNK, zrow, 0)
        for kpart in range(ROWS_PER_TILE // CHUNK):
            pltpu.sync_copy(
                rows0, agg.at[pl.ds(s * ROWS_PER_TILE + kpart * CHUNK, CHUNK)])
        zrem = ROWS_PER_TILE % CHUNK
        if zrem:
            pltpu.sync_copy(
                rows0.at[pl.ds(0, zrem)],
                agg.at[pl.ds(s * ROWS_PER_TILE + ROWS_PER_TILE - zrem, zrem)])

        @pl.when(s == NS - 1)
        def _zero_tail():
            pltpu.sync_copy(
                rows0.at[pl.ds(0, ROWS_TAIL)],
                agg.at[pl.ds(NS * ROWS_PER_TILE, ROWS_TAIL)])

        plsc.subcore_barrier()

        def gfire(j, buf, gsem):
            pltpu.async_copy(
                h_hbm.at[sidx.at[pl.ds(j * CHUNK, CHUNK)]], buf, gsem)

        def gwait(buf, gsem):
            pltpu.make_async_copy(
                h_hbm.at[sidx.at[pl.ds(0, CHUNK)]], buf, gsem).wait()

        def rfire(j, rbuf, rsem):
            pltpu.async_copy(
                ei_hbm.at[pl.ds(rbase + j * CHUNK, CHUNK)], rbuf, rsem)

        def rwait(rbuf, rsem):
            pltpu.make_async_copy(
                ei_hbm.at[pl.ds(0, CHUNK)], rbuf, rsem).wait()

        def fire(j, k):
            gfire(j, rowss[k], gsems[k])
            rfire(j, ridxs[k], rsems[k])

        def waitb(k):
            gwait(rowss[k], gsems[k])
            rwait(ridxs[k], rsems[k])

        def scatb(k):
            pltpu.sync_copy(rowss[k], agg.at[ridxs[k]], add=True)

        for k in range(NBUF - 1):
            fire(k, k)

        def chunk_quad(i, carry):
            j = NBUF * i
            for k in range(NBUF):
                fire(j + NBUF - 1 + k, (NBUF - 1 + k) % NBUF)
                waitb(k)
                scatb(k)
            return carry

        lax.fori_loop(0, FULL // NBUF - 1, chunk_quad, 0)
        fire(FULL - 1, NBUF - 1)
        for k in range(NBUF):
            waitb(k)
            scatb(k)
        plsc.subcore_barrier()

        pltpu.sync_copy(
            agg.at[pl.ds(s * ROWS_PER_TILE, ROWS_PER_TILE)],
            out_hbm.at[c, pl.ds(s * ROWS_PER_TILE, ROWS_PER_TILE)])

        @pl.when(s == NS - 1)
        def _copy_tail():
            pltpu.sync_copy(
                agg.at[pl.ds(NS * ROWS_PER_TILE, ROWS_TAIL)],
                out_hbm.at[c, pl.ds(NS * ROWS_PER_TILE, ROWS_TAIL)])

    return agg_kernel(h, ei)


def _mlp_kernel(h_ref, a_ref, w1_ref, b1_ref, w2_ref, b2_ref, o_ref):
    x = h_ref[...] + a_ref[0] + a_ref[1]
    z = lax.dot_general(
        x, w1_ref[...], dimension_numbers=(((1,), (1,)), ((), ())),
        preferred_element_type=jnp.float32,
        precision=lax.Precision.HIGHEST) + b1_ref[...]
    z = jnp.maximum(z, 0.0)
    z = lax.dot_general(
        z, w2_ref[...], dimension_numbers=(((1,), (1,)), ((), ())),
        preferred_element_type=jnp.float32,
        precision=lax.Precision.HIGHEST) + b2_ref[...]
    o_ref[...] = z


def kernel(h, edge_index, W1, b1, W2, b2):
    ei = edge_index.astype(jnp.int32).reshape(2 * E)
    agg = _sc_aggregate(h, ei)
    grid = N_NODES // MLP_BLOCK
    out = pl.pallas_call(
        _mlp_kernel,
        grid=(grid,),
        in_specs=[
            pl.BlockSpec((MLP_BLOCK, D), lambda i: (i, 0)),
            pl.BlockSpec((NC, MLP_BLOCK, D), lambda i: (0, i, 0)),
            pl.BlockSpec((D, D), lambda i: (0, 0)),
            pl.BlockSpec((1, D), lambda i: (0, 0)),
            pl.BlockSpec((D, D), lambda i: (0, 0)),
            pl.BlockSpec((1, D), lambda i: (0, 0)),
        ],
        out_specs=pl.BlockSpec((MLP_BLOCK, D), lambda i: (i, 0)),
        out_shape=jax.ShapeDtypeStruct((N_NODES, D), jnp.float32),
    )(h, agg, W1, b1.reshape(1, D), W2, b2.reshape(1, D))
    return out


# speedup vs baseline: 1.0070x; 1.0070x over previous
import functools

import jax
import jax.numpy as jnp
from jax import lax
from jax.experimental import pallas as pl
from jax.experimental.pallas import tpu as pltpu
from jax.experimental.pallas import tpu_sc as plsc

N_NODES = 10000
D = 128
E = 320000
NC = 2
NS = 16
NW = NC * NS
E_PER_TILE = E // NW
CHUNK = 64
FULL = E_PER_TILE // CHUNK
NBUF = 4
TAIL = E_PER_TILE - FULL * CHUNK
ROWS_PER_TILE = 624
ROWS_TAIL = N_NODES - NS * ROWS_PER_TILE
MLP_BLOCK = 2000


def _sc_aggregate(h, ei):
    mesh = plsc.VectorSubcoreMesh(core_axis_name="c", subcore_axis_name="s")

    @functools.partial(
        pl.kernel,
        mesh=mesh,
        out_type=jax.ShapeDtypeStruct((NC, N_NODES, D), jnp.float32),
        scratch_types=[
            pltpu.VMEM((E_PER_TILE,), jnp.int32),
            *[pltpu.VMEM((CHUNK,), jnp.int32) for _ in range(NBUF)],
            *[pltpu.VMEM((CHUNK, D), jnp.float32) for _ in range(NBUF)],
            pltpu.VMEM_SHARED((N_NODES, D), jnp.float32),
            *[pltpu.SemaphoreType.DMA for _ in range(2 * NBUF)],
            *([pltpu.VMEM((TAIL,), jnp.int32),
               pltpu.VMEM((TAIL, D), jnp.float32),
               pltpu.SemaphoreType.DMA,
               pltpu.SemaphoreType.DMA] if TAIL else []),
        ],
    )
    def agg_kernel(h_hbm, ei_hbm, out_hbm, *scr):
        sidx = scr[0]
        ridxs = scr[1:1 + NBUF]
        rowss = scr[1 + NBUF:1 + 2 * NBUF]
        agg = scr[1 + 2 * NBUF]
        gsems = scr[2 + 2 * NBUF:2 + 3 * NBUF]
        rsems = scr[2 + 3 * NBUF:2 + 4 * NBUF]
        if TAIL:
            ridxt, rowst, gsemt, rsemt = scr[2 + 4 * NBUF:6 + 4 * NBUF]
        rows0 = rowss[0]
        c = lax.axis_index("c")
        s = lax.axis_index("s")
        w = c * NS + s
        ebase = w * E_PER_TILE
        rbase = E + ebase

        pltpu.sync_copy(ei_hbm.at[pl.ds(ebase, E_PER_TILE)], sidx)

        zero = jnp.zeros((16,), jnp.float32)

        def zrow(r, carry):
            for cc in range(D // 16):
                rows0[r, pl.ds(cc * 16, 16)] = zero
            return carry

        lax.fori_loop(0, CHUNK, zrow, 0)
        for kpart in range(ROWS_PER_TILE // CHUNK):
            pltpu.sync_copy(
                rows0, agg.at[pl.ds(s * ROWS_PER_TILE + kpart * CHUNK, CHUNK)])
        zrem = ROWS_PER_TILE % CHUNK
        if zrem:
            pltpu.sync_copy(
                rows0.at[pl.ds(0, zrem)],
                agg.at[pl.ds(s * ROWS_PER_TILE + ROWS_PER_TILE - zrem, zrem)])

        @pl.when(s == NS - 1)
        def _zero_tail():
            pltpu.sync_copy(
                rows0.at[pl.ds(0, ROWS_TAIL)],
                agg.at[pl.ds(NS * ROWS_PER_TILE, ROWS_TAIL)])

        plsc.subcore_barrier()

        def gfire(j, buf, gsem):
            pltpu.async_copy(
                h_hbm.at[sidx.at[pl.ds(j * CHUNK, CHUNK)]], buf, gsem)

        def gwait(buf, gsem):
            pltpu.make_async_copy(
                h_hbm.at[sidx.at[pl.ds(0, CHUNK)]], buf, gsem).wait()

        def rfire(j, rbuf, rsem):
            pltpu.async_copy(
                ei_hbm.at[pl.ds(rbase + j * CHUNK, CHUNK)], rbuf, rsem)

        def rwait(rbuf, rsem):
            pltpu.make_async_copy(
                ei_hbm.at[pl.ds(0, CHUNK)], rbuf, rsem).wait()

        def fire(j, k):
            gfire(j, rowss[k], gsems[k])
            rfire(j, ridxs[k], rsems[k])

        def waitb(k):
            gwait(rowss[k], gsems[k])
            rwait(ridxs[k], rsems[k])

        def scatb(k):
            pltpu.sync_copy(rowss[k], agg.at[ridxs[k]], add=True)

        for k in range(NBUF - 1):
            fire(k, k)

        def chunk_quad(i, carry):
            j = NBUF * i
            for k in range(NBUF):
                fire(j + NBUF - 1 + k, (NBUF - 1 + k) % NBUF)
                waitb(k)
                scatb(k)
            return carry

        lax.fori_loop(0, FULL // NBUF - 1, chunk_quad, 0)
        fire(FULL - 1, NBUF - 1)
        waitb(0)
        scatb(0)
        if TAIL:
            pltpu.async_copy(
                h_hbm.at[sidx.at[pl.ds(FULL * CHUNK, TAIL)]], rowst, gsemt)
            pltpu.async_copy(
                ei_hbm.at[pl.ds(rbase + FULL * CHUNK, TAIL)], ridxt, rsemt)
        for k in range(1, NBUF):
            waitb(k)
            scatb(k)
        if TAIL:
            pltpu.make_async_copy(
                h_hbm.at[sidx.at[pl.ds(0, TAIL)]], rowst, gsemt).wait()
            pltpu.make_async_copy(
                ei_hbm.at[pl.ds(0, TAIL)], ridxt, rsemt).wait()
            pltpu.sync_copy(rowst, agg.at[ridxt], add=True)
        plsc.subcore_barrier()

        pltpu.sync_copy(
            agg.at[pl.ds(s * ROWS_PER_TILE, ROWS_PER_TILE)],
            out_hbm.at[c, pl.ds(s * ROWS_PER_TILE, ROWS_PER_TILE)])

        @pl.when(s == NS - 1)
        def _copy_tail():
            pltpu.sync_copy(
                agg.at[pl.ds(NS * ROWS_PER_TILE, ROWS_TAIL)],
                out_hbm.at[c, pl.ds(NS * ROWS_PER_TILE, ROWS_TAIL)])

    return agg_kernel(h, ei)


def _mlp_kernel(h_ref, a_ref, w1_ref, b1_ref, w2_ref, b2_ref, o_ref):
    x = h_ref[...] + a_ref[0] + a_ref[1]
    z = lax.dot_general(
        x, w1_ref[...], dimension_numbers=(((1,), (1,)), ((), ())),
        preferred_element_type=jnp.float32,
        precision=lax.Precision.HIGHEST) + b1_ref[...]
    z = jnp.maximum(z, 0.0)
    z = lax.dot_general(
        z, w2_ref[...], dimension_numbers=(((1,), (1,)), ((), ())),
        preferred_element_type=jnp.float32,
        precision=lax.Precision.HIGHEST) + b2_ref[...]
    o_ref[...] = z


def kernel(h, edge_index, W1, b1, W2, b2):
    ei = edge_index.astype(jnp.int32).reshape(2 * E)
    agg = _sc_aggregate(h, ei)
    grid = N_NODES // MLP_BLOCK
    out = pl.pallas_call(
        _mlp_kernel,
        grid=(grid,),
        in_specs=[
            pl.BlockSpec((MLP_BLOCK, D), lambda i: (i, 0)),
            pl.BlockSpec((NC, MLP_BLOCK, D), lambda i: (0, i, 0)),
            pl.BlockSpec((D, D), lambda i: (0, 0)),
            pl.BlockSpec((1, D), lambda i: (0, 0)),
            pl.BlockSpec((D, D), lambda i: (0, 0)),
            pl.BlockSpec((1, D), lambda i: (0, 0)),
        ],
        out_specs=pl.BlockSpec((MLP_BLOCK, D), lambda i: (i, 0)),
        out_shape=jax.ShapeDtypeStruct((N_NODES, D), jnp.float32),
    )(h, agg, W1, b1.reshape(1, D), W2, b2.reshape(1, D))
    return out


# speedup vs baseline: 1.0081x; 1.0011x over previous
import functools

import jax
import jax.numpy as jnp
from jax import lax
from jax.experimental import pallas as pl
from jax.experimental.pallas import tpu as pltpu
from jax.experimental.pallas import tpu_sc as plsc

N_NODES = 10000
D = 128
E = 320000
NC = 2
NS = 16
NW = NC * NS
E_PER_TILE = E // NW
CHUNK = 64
FULL = E_PER_TILE // CHUNK
NBUF = 4
TAIL = E_PER_TILE - FULL * CHUNK
ROWS_PER_TILE = 624
ROWS_TAIL = N_NODES - NS * ROWS_PER_TILE
MLP_BLOCK = 2000


def _sc_aggregate(h, ei):
    mesh = plsc.VectorSubcoreMesh(core_axis_name="c", subcore_axis_name="s")

    @functools.partial(
        pl.kernel,
        mesh=mesh,
        out_type=jax.ShapeDtypeStruct((NC, N_NODES, D), jnp.float32),
        scratch_types=[
            pltpu.VMEM((E_PER_TILE,), jnp.int32),
            *[pltpu.VMEM((CHUNK,), jnp.int32) for _ in range(NBUF)],
            *[pltpu.VMEM((CHUNK, D), jnp.float32) for _ in range(NBUF)],
            pltpu.VMEM_SHARED((N_NODES, D), jnp.float32),
            *[pltpu.SemaphoreType.DMA for _ in range(2 * NBUF)],
            *([pltpu.VMEM((TAIL,), jnp.int32),
               pltpu.VMEM((TAIL, D), jnp.float32),
               pltpu.SemaphoreType.DMA,
               pltpu.SemaphoreType.DMA] if TAIL else []),
        ],
        compiler_params=pltpu.CompilerParams(use_tc_tiling_on_sc=False),
    )
    def agg_kernel(h_hbm, ei_hbm, out_hbm, *scr):
        sidx = scr[0]
        ridxs = scr[1:1 + NBUF]
        rowss = scr[1 + NBUF:1 + 2 * NBUF]
        agg = scr[1 + 2 * NBUF]
        gsems = scr[2 + 2 * NBUF:2 + 3 * NBUF]
        rsems = scr[2 + 3 * NBUF:2 + 4 * NBUF]
        if TAIL:
            ridxt, rowst, gsemt, rsemt = scr[2 + 4 * NBUF:6 + 4 * NBUF]
        rows0 = rowss[0]
        c = lax.axis_index("c")
        s = lax.axis_index("s")
        w = c * NS + s
        ebase = w * E_PER_TILE

        pltpu.sync_copy(ei_hbm.at[0, pl.ds(ebase, E_PER_TILE)], sidx)

        zero = jnp.zeros((16,), jnp.float32)

        def zrow(r, carry):
            for cc in range(D // 16):
                rows0[r, pl.ds(cc * 16, 16)] = zero
            return carry

        lax.fori_loop(0, CHUNK, zrow, 0)
        for kpart in range(ROWS_PER_TILE // CHUNK):
            pltpu.sync_copy(
                rows0, agg.at[pl.ds(s * ROWS_PER_TILE + kpart * CHUNK, CHUNK)])
        zrem = ROWS_PER_TILE % CHUNK
        if zrem:
            pltpu.sync_copy(
                rows0.at[pl.ds(0, zrem)],
                agg.at[pl.ds(s * ROWS_PER_TILE + ROWS_PER_TILE - zrem, zrem)])

        @pl.when(s == NS - 1)
        def _zero_tail():
            pltpu.sync_copy(
                rows0.at[pl.ds(0, ROWS_TAIL)],
                agg.at[pl.ds(NS * ROWS_PER_TILE, ROWS_TAIL)])

        plsc.subcore_barrier()

        def gfire(j, buf, gsem):
            pltpu.async_copy(
                h_hbm.at[sidx.at[pl.ds(j * CHUNK, CHUNK)]], buf, gsem)

        def gwait(buf, gsem):
            pltpu.make_async_copy(
                h_hbm.at[sidx.at[pl.ds(0, CHUNK)]], buf, gsem).wait()

        def rfire(j, rbuf, rsem):
            pltpu.async_copy(
                ei_hbm.at[1, pl.ds(ebase + j * CHUNK, CHUNK)], rbuf, rsem)

        def rwait(rbuf, rsem):
            pltpu.make_async_copy(
                ei_hbm.at[1, pl.ds(0, CHUNK)], rbuf, rsem).wait()

        def fire(j, k):
            gfire(j, rowss[k], gsems[k])
            rfire(j, ridxs[k], rsems[k])

        def waitb(k):
            gwait(rowss[k], gsems[k])
            rwait(ridxs[k], rsems[k])

        def scatb(k):
            pltpu.sync_copy(rowss[k], agg.at[ridxs[k]], add=True)

        for k in range(NBUF - 1):
            fire(k, k)

        def chunk_quad(i, carry):
            j = NBUF * i
            for k in range(NBUF):
                fire(j + NBUF - 1 + k, (NBUF - 1 + k) % NBUF)
                waitb(k)
                scatb(k)
            return carry

        lax.fori_loop(0, FULL // NBUF - 1, chunk_quad, 0)
        fire(FULL - 1, NBUF - 1)
        waitb(0)
        scatb(0)
        if TAIL:
            pltpu.async_copy(
                h_hbm.at[sidx.at[pl.ds(FULL * CHUNK, TAIL)]], rowst, gsemt)
            pltpu.async_copy(
                ei_hbm.at[1, pl.ds(ebase + FULL * CHUNK, TAIL)], ridxt, rsemt)
        for k in range(1, NBUF):
            waitb(k)
            scatb(k)
        if TAIL:
            pltpu.make_async_copy(
                h_hbm.at[sidx.at[pl.ds(0, TAIL)]], rowst, gsemt).wait()
            pltpu.make_async_copy(
                ei_hbm.at[1, pl.ds(0, TAIL)], ridxt, rsemt).wait()
            pltpu.sync_copy(rowst, agg.at[ridxt], add=True)
        plsc.subcore_barrier()

        pltpu.sync_copy(
            agg.at[pl.ds(s * ROWS_PER_TILE, ROWS_PER_TILE)],
            out_hbm.at[c, pl.ds(s * ROWS_PER_TILE, ROWS_PER_TILE)])

        @pl.when(s == NS - 1)
        def _copy_tail():
            pltpu.sync_copy(
                agg.at[pl.ds(NS * ROWS_PER_TILE, ROWS_TAIL)],
                out_hbm.at[c, pl.ds(NS * ROWS_PER_TILE, ROWS_TAIL)])

    return agg_kernel(h, ei)


def _mlp_kernel(h_ref, a_ref, w1_ref, b1_ref, w2_ref, b2_ref, o_ref):
    x = h_ref[...] + a_ref[0] + a_ref[1]
    z = lax.dot_general(
        x, w1_ref[...], dimension_numbers=(((1,), (1,)), ((), ())),
        preferred_element_type=jnp.float32,
        precision=lax.Precision.HIGHEST) + b1_ref[...]
    z = jnp.maximum(z, 0.0)
    z = lax.dot_general(
        z, w2_ref[...], dimension_numbers=(((1,), (1,)), ((), ())),
        preferred_element_type=jnp.float32,
        precision=lax.Precision.HIGHEST) + b2_ref[...]
    o_ref[...] = z


def kernel(h, edge_index, W1, b1, W2, b2):
    ei = edge_index.astype(jnp.int32)
    agg = _sc_aggregate(h, ei)
    grid = N_NODES // MLP_BLOCK
    out = pl.pallas_call(
        _mlp_kernel,
        grid=(grid,),
        in_specs=[
            pl.BlockSpec((MLP_BLOCK, D), lambda i: (i, 0)),
            pl.BlockSpec((NC, MLP_BLOCK, D), lambda i: (0, i, 0)),
            pl.BlockSpec((D, D), lambda i: (0, 0)),
            pl.BlockSpec((1, D), lambda i: (0, 0)),
            pl.BlockSpec((D, D), lambda i: (0, 0)),
            pl.BlockSpec((1, D), lambda i: (0, 0)),
        ],
        out_specs=pl.BlockSpec((MLP_BLOCK, D), lambda i: (i, 0)),
        out_shape=jax.ShapeDtypeStruct((N_NODES, D), jnp.float32),
    )(h, agg, W1, b1.reshape(1, D), W2, b2.reshape(1, D))
    return out
